# Initial kernel scaffold; baseline (speedup 1.0000x reference)
import jax
import jax.numpy as jnp
from jax.experimental import pallas as pl


def kernel(x, edge_index, local_topo, global_topo, W1, b1, attW1, attb1, attw2, Wg, bg):
    raise NotImplementedError("write your pallas kernel here")


# speedup vs baseline: 8.6351x; 8.6351x over previous
import functools

import jax
import jax.numpy as jnp
from jax import lax
from jax.experimental import pallas as pl
from jax.experimental.pallas import tpu as pltpu
from jax.experimental.pallas import tpu_sc as plsc

ALPHA = 0.1
K = 10
NC, NS = 2, 16
NW = NC * NS
EB = 128


def _mesh():
    return plsc.VectorSubcoreMesh(
        core_axis_name="c", subcore_axis_name="s",
        num_cores=NC, num_subcores=NS)


_SC_PARAMS = pltpu.CompilerParams(needs_layout_passes=False,
                                  use_tc_tiling_on_sc=False)


@functools.lru_cache(maxsize=None)
def _make_prop(n_pad, bpt, width):
    rps = n_pad // NS

    def body(u_hbm, sidx_hbm, didx_hbm, part_hbm,
             sidx_v, didx_v, rows_v, zrows_v, acc_sh):
        cid = lax.axis_index("c")
        sid = lax.axis_index("s")
        wid = sid * NC + cid
        pltpu.sync_copy(sidx_hbm.at[wid], sidx_v)
        pltpu.sync_copy(didx_hbm.at[wid], didx_v)
        zero = jnp.zeros((16,), jnp.float32)

        def zb(i, c):
            for j in range(width // 16):
                zrows_v[i, pl.ds(j * 16, 16)] = zero
            return c
        lax.fori_loop(0, rps, zb, 0)
        pltpu.sync_copy(zrows_v, acc_sh.at[pl.ds(sid * rps, rps)])
        plsc.subcore_barrier()

        def eb(b, c):
            pltpu.sync_copy(u_hbm.at[sidx_v.at[b]], rows_v)
            pltpu.sync_copy(rows_v, acc_sh.at[didx_v.at[b]], add=True)
            return c
        lax.fori_loop(0, bpt, eb, 0)
        plsc.subcore_barrier()
        pltpu.sync_copy(acc_sh.at[pl.ds(sid * rps, rps)],
                        part_hbm.at[cid, pl.ds(sid * rps, rps)])

    return pl.kernel(
        body,
        out_type=jax.ShapeDtypeStruct((NC, n_pad, width), jnp.float32),
        mesh=_mesh(),
        compiler_params=_SC_PARAMS,
        scratch_types=[
            pltpu.VMEM((bpt, EB), jnp.int32),
            pltpu.VMEM((bpt, EB), jnp.int32),
            pltpu.VMEM((EB, width), jnp.float32),
            pltpu.VMEM((rps, width), jnp.float32),
            pltpu.VMEM_SHARED((n_pad, width), jnp.float32),
        ],
    )


@functools.lru_cache(maxsize=None)
def _make_deg(n_pad, bpt):
    def body(didx_hbm, degp_hbm, didx_v, deg_v):
        cid = lax.axis_index("c")
        sid = lax.axis_index("s")
        wid = sid * NC + cid
        pltpu.sync_copy(didx_hbm.at[wid], didx_v)
        zero = jnp.zeros((16,), jnp.float32)

        def zb(i, c):
            deg_v[pl.ds(i * 16, 16)] = zero
            return c
        lax.fori_loop(0, n_pad // 16, zb, 0)
        ones = jnp.ones((16,), jnp.float32)

        def eb(b, c):
            for g in range(EB // 16):
                idx = didx_v[b, pl.ds(g * 16, 16)]
                plsc.addupdate_scatter(deg_v, [idx], ones)
            return c
        lax.fori_loop(0, bpt, eb, 0)
        pltpu.sync_copy(deg_v, degp_hbm.at[wid])

    return pl.kernel(
        body,
        out_type=jax.ShapeDtypeStruct((NW, n_pad), jnp.float32),
        mesh=_mesh(),
        compiler_params=_SC_PARAMS,
        scratch_types=[
            pltpu.VMEM((bpt, EB), jnp.int32),
            pltpu.VMEM((n_pad,), jnp.float32),
        ],
    )


def _mm1_body(x_ref, w_ref, b_ref, h_ref):
    h_ref[...] = jnp.maximum(
        jnp.dot(x_ref[...], w_ref[...], preferred_element_type=jnp.float32)
        + b_ref[...], 0.0)


def _dinv_body(degt_ref, h_ref, dinv_ref, u0_ref, *, n_real):
    deg = jnp.sum(degt_ref[...], axis=1, keepdims=True) + 1.0
    dinv = lax.rsqrt(deg)
    row = lax.broadcasted_iota(jnp.int32, dinv.shape, 0)
    dinv = jnp.where(row < n_real, dinv, 0.0)
    dinv_ref[...] = dinv
    u0_ref[...] = dinv * h_ref[...]


def _combine_body(part_ref, u_ref, h_ref, dinv_ref, z_ref, un_ref):
    dinv = dinv_ref[...]
    u = u_ref[...]
    z = (1.0 - ALPHA) * dinv * (part_ref[0] + part_ref[1] + u) \
        + ALPHA * h_ref[...]
    z_ref[...] = z
    un_ref[...] = dinv * z


def _att_body(z_ref, lt_ref, g_ref, aw1_ref, ab1_ref, aw2_ref, wg_ref,
              dinv_ref, v_ref):
    z = z_ref[...]
    lt = lt_ref[...]
    aw1 = aw1_ref[...]
    ab1 = ab1_ref[...]
    aw2 = aw2_ref[...]
    wz = jnp.dot(jnp.tanh(
        jnp.dot(z, aw1, preferred_element_type=jnp.float32) + ab1),
        aw2, preferred_element_type=jnp.float32)
    wl = jnp.dot(jnp.tanh(
        jnp.dot(lt, aw1, preferred_element_type=jnp.float32) + ab1),
        aw2, preferred_element_type=jnp.float32)
    m = jnp.maximum(wz, wl)
    ez = jnp.exp(wz - m)
    el = jnp.exp(wl - m)
    emb2 = (ez * z + el * lt) / (ez + el)
    z2 = emb2 * g_ref[...]
    v_ref[...] = dinv_ref[...] * jnp.dot(
        z2, wg_ref[...], preferred_element_type=jnp.float32)


def _final_body(part_ref, v_ref, dinv_ref, bg_ref, out_ref):
    o = dinv_ref[...] * (part_ref[0] + part_ref[1] + v_ref[...]) + bg_ref[...]
    m = jnp.max(o, axis=1, keepdims=True)
    s = jnp.sum(jnp.exp(o - m), axis=1, keepdims=True)
    out_ref[...] = (o - m) - jnp.log(s)


def kernel(x, edge_index, local_topo, global_topo, W1, b1,
           attW1, attb1, attw2, Wg, bg):
    n, nfeat = x.shape
    e = edge_index.shape[1]
    nhid = W1.shape[1]
    nclass = Wg.shape[1]
    n_pad = -(-(n + 1) // (NS * 8)) * NS * 8
    bpt = -(-e // (NW * EB))
    bpt += bpt % 2
    e_pad = NW * EB * bpt

    src = edge_index[0]
    dst = edge_index[1]
    fill = jnp.full((e_pad - e,), n, jnp.int32)
    sidx = jnp.concatenate([src, fill]).reshape(NW, bpt, EB)
    didx = jnp.concatenate([dst, fill]).reshape(NW, bpt, EB)

    pad_n = n_pad - n
    xp = jnp.pad(x, ((0, pad_n), (0, 0)))
    ltp = jnp.pad(local_topo, ((0, pad_n), (0, 0)))
    b1r = b1.reshape(1, nhid)
    ab1r = attb1.reshape(1, -1)
    bgr = bg.reshape(1, nclass)
    g = global_topo.reshape(1, nhid)

    f32 = jnp.float32
    h = pl.pallas_call(
        _mm1_body,
        out_shape=jax.ShapeDtypeStruct((n_pad, nhid), f32))(xp, W1, b1r)

    degp = _make_deg(n_pad, bpt)(didx)
    dinv, u = pl.pallas_call(
        functools.partial(_dinv_body, n_real=n),
        out_shape=[jax.ShapeDtypeStruct((n_pad, 1), f32),
                   jax.ShapeDtypeStruct((n_pad, nhid), f32)])(degp.T, h)

    prop = _make_prop(n_pad, bpt, nhid)
    z = h
    for _ in range(K):
        part = prop(u, sidx, didx)
        z, u = pl.pallas_call(
            _combine_body,
            out_shape=[jax.ShapeDtypeStruct((n_pad, nhid), f32),
                       jax.ShapeDtypeStruct((n_pad, nhid), f32)])(
            part, u, h, dinv)

    v = pl.pallas_call(
        _att_body,
        out_shape=jax.ShapeDtypeStruct((n_pad, nclass), f32))(
        z, ltp, g, attW1, ab1r, attw2, Wg, dinv)

    partf = _make_prop(n_pad, bpt, nclass)(v, sidx, didx)
    out = pl.pallas_call(
        _final_body,
        out_shape=jax.ShapeDtypeStruct((n_pad, nclass), f32))(
        partf, v, dinv, bgr)
    return out[:n]


# speedup vs baseline: 10.1337x; 1.1735x over previous
import functools

import jax
import jax.numpy as jnp
from jax import lax
from jax.experimental import pallas as pl
from jax.experimental.pallas import tpu as pltpu
from jax.experimental.pallas import tpu_sc as plsc

ALPHA = 0.1
K = 10
NC, NS = 2, 16
NW = NC * NS
EB = 128


def _mesh():
    return plsc.VectorSubcoreMesh(
        core_axis_name="c", subcore_axis_name="s",
        num_cores=NC, num_subcores=NS)


_SC_PARAMS = pltpu.CompilerParams(needs_layout_passes=False,
                                  use_tc_tiling_on_sc=False)


DEPTH = 4


@functools.lru_cache(maxsize=None)
def _make_prop(n_pad, bpt, width):
    rps = n_pad // NS
    zr = rps // 4 if rps % 4 == 0 else rps

    def body(u_hbm, sidx_hbm, didx_hbm, part_hbm,
             sidx_v, didx_v, zrows_v, acc_sh, *bufs):
        rows = bufs[:DEPTH]
        semg = bufs[DEPTH:2 * DEPTH]
        sems = bufs[2 * DEPTH:3 * DEPTH]
        cid = lax.axis_index("c")
        sid = lax.axis_index("s")
        wid = sid * NC + cid
        pltpu.sync_copy(sidx_hbm.at[wid], sidx_v)
        pltpu.sync_copy(didx_hbm.at[wid], didx_v)
        zero = jnp.zeros((16,), jnp.float32)

        def zb(i, c):
            for j in range(width // 16):
                zrows_v[i, pl.ds(j * 16, 16)] = zero
            return c
        lax.fori_loop(0, zr, zb, 0)

        def zc(i, c):
            pltpu.sync_copy(zrows_v, acc_sh.at[pl.ds(sid * rps + i * zr, zr)])
            return c
        lax.fori_loop(0, rps // zr, zc, 0)
        plsc.subcore_barrier()

        for j in range(DEPTH):
            pltpu.async_copy(u_hbm.at[sidx_v.at[j]], rows[j], semg[j])

        def eb(i, c):
            b0 = i * DEPTH
            for j in range(DEPTH):
                b = b0 + j
                pltpu.make_async_copy(
                    u_hbm.at[sidx_v.at[b]], rows[j], semg[j]).wait()
                pltpu.async_copy(
                    rows[j], acc_sh.at[didx_v.at[b]], sems[j], add=True)
            for j in range(DEPTH):
                b = b0 + j
                pltpu.make_async_copy(
                    rows[j], acc_sh.at[didx_v.at[b]], sems[j]).wait()
                nb = b0 + DEPTH + j

                @pl.when(nb < bpt)
                def _():
                    pltpu.async_copy(u_hbm.at[sidx_v.at[nb]],
                                     rows[j], semg[j])
            return c
        lax.fori_loop(0, bpt // DEPTH, eb, 0)
        plsc.subcore_barrier()
        pltpu.sync_copy(acc_sh.at[pl.ds(sid * rps, rps)],
                        part_hbm.at[cid, pl.ds(sid * rps, rps)])

    return pl.kernel(
        body,
        out_type=jax.ShapeDtypeStruct((NC, n_pad, width), jnp.float32),
        mesh=_mesh(),
        compiler_params=_SC_PARAMS,
        scratch_types=[
            pltpu.VMEM((bpt, EB), jnp.int32),
            pltpu.VMEM((bpt, EB), jnp.int32),
            pltpu.VMEM((zr, width), jnp.float32),
            pltpu.VMEM_SHARED((n_pad, width), jnp.float32),
        ] + [pltpu.VMEM((EB, width), jnp.float32)] * DEPTH
          + [pltpu.SemaphoreType.DMA] * (2 * DEPTH),
    )


@functools.lru_cache(maxsize=None)
def _make_deg(n_pad, bpt):
    def body(didx_hbm, degp_hbm, didx_v, deg_v):
        cid = lax.axis_index("c")
        sid = lax.axis_index("s")
        wid = sid * NC + cid
        pltpu.sync_copy(didx_hbm.at[wid], didx_v)
        zero = jnp.zeros((16,), jnp.float32)

        def zb(i, c):
            deg_v[pl.ds(i * 16, 16)] = zero
            return c
        lax.fori_loop(0, n_pad // 16, zb, 0)
        ones = jnp.ones((16,), jnp.float32)

        def eb(b, c):
            for g in range(EB // 16):
                idx = didx_v[b, pl.ds(g * 16, 16)]
                plsc.addupdate_scatter(deg_v, [idx], ones)
            return c
        lax.fori_loop(0, bpt, eb, 0)
        pltpu.sync_copy(deg_v, degp_hbm.at[wid])

    return pl.kernel(
        body,
        out_type=jax.ShapeDtypeStruct((NW, n_pad), jnp.float32),
        mesh=_mesh(),
        compiler_params=_SC_PARAMS,
        scratch_types=[
            pltpu.VMEM((bpt, EB), jnp.int32),
            pltpu.VMEM((n_pad,), jnp.float32),
        ],
    )


def _mm1_body(x_ref, w_ref, b_ref, h_ref):
    h_ref[...] = jnp.maximum(
        jnp.dot(x_ref[...], w_ref[...], preferred_element_type=jnp.float32)
        + b_ref[...], 0.0)


def _dinv_body(degt_ref, h_ref, dinv_ref, u0_ref, *, n_real):
    deg = jnp.sum(degt_ref[...], axis=1, keepdims=True) + 1.0
    dinv = lax.rsqrt(deg)
    row = lax.broadcasted_iota(jnp.int32, dinv.shape, 0)
    dinv = jnp.where(row < n_real, dinv, 0.0)
    dinv_ref[...] = dinv
    u0_ref[...] = dinv * h_ref[...]


def _combine_body(part_ref, u_ref, h_ref, dinv_ref, z_ref, un_ref):
    dinv = dinv_ref[...]
    u = u_ref[...]
    z = (1.0 - ALPHA) * dinv * (part_ref[0] + part_ref[1] + u) \
        + ALPHA * h_ref[...]
    z_ref[...] = z
    un_ref[...] = dinv * z


def _att_body(z_ref, lt_ref, g_ref, aw1_ref, ab1_ref, aw2_ref, wg_ref,
              dinv_ref, v_ref):
    z = z_ref[...]
    lt = lt_ref[...]
    aw1 = aw1_ref[...]
    ab1 = ab1_ref[...]
    aw2 = aw2_ref[...]
    wz = jnp.dot(jnp.tanh(
        jnp.dot(z, aw1, preferred_element_type=jnp.float32) + ab1),
        aw2, preferred_element_type=jnp.float32)
    wl = jnp.dot(jnp.tanh(
        jnp.dot(lt, aw1, preferred_element_type=jnp.float32) + ab1),
        aw2, preferred_element_type=jnp.float32)
    m = jnp.maximum(wz, wl)
    ez = jnp.exp(wz - m)
    el = jnp.exp(wl - m)
    emb2 = (ez * z + el * lt) / (ez + el)
    z2 = emb2 * g_ref[...]
    v_ref[...] = dinv_ref[...] * jnp.dot(
        z2, wg_ref[...], preferred_element_type=jnp.float32)


def _final_body(part_ref, v_ref, dinv_ref, bg_ref, out_ref):
    o = dinv_ref[...] * (part_ref[0] + part_ref[1] + v_ref[...]) + bg_ref[...]
    m = jnp.max(o, axis=1, keepdims=True)
    s = jnp.sum(jnp.exp(o - m), axis=1, keepdims=True)
    out_ref[...] = (o - m) - jnp.log(s)


def kernel(x, edge_index, local_topo, global_topo, W1, b1,
           attW1, attb1, attw2, Wg, bg):
    n, nfeat = x.shape
    e = edge_index.shape[1]
    nhid = W1.shape[1]
    nclass = Wg.shape[1]
    n_pad = -(-(n + 1) // (NS * 8)) * NS * 8
    bpt = -(-e // (NW * EB))
    bpt = -(-bpt // DEPTH) * DEPTH
    e_pad = NW * EB * bpt

    src = edge_index[0]
    dst = edge_index[1]
    fill = jnp.full((e_pad - e,), n, jnp.int32)
    sidx = jnp.concatenate([src, fill]).reshape(NW, bpt, EB)
    didx = jnp.concatenate([dst, fill]).reshape(NW, bpt, EB)

    pad_n = n_pad - n
    xp = jnp.pad(x, ((0, pad_n), (0, 0)))
    ltp = jnp.pad(local_topo, ((0, pad_n), (0, 0)))
    b1r = b1.reshape(1, nhid)
    ab1r = attb1.reshape(1, -1)
    bgr = bg.reshape(1, nclass)
    g = global_topo.reshape(1, nhid)

    f32 = jnp.float32
    h = pl.pallas_call(
        _mm1_body,
        out_shape=jax.ShapeDtypeStruct((n_pad, nhid), f32))(xp, W1, b1r)

    degp = _make_deg(n_pad, bpt)(didx)
    dinv, u = pl.pallas_call(
        functools.partial(_dinv_body, n_real=n),
        out_shape=[jax.ShapeDtypeStruct((n_pad, 1), f32),
                   jax.ShapeDtypeStruct((n_pad, nhid), f32)])(degp.T, h)

    prop = _make_prop(n_pad, bpt, nhid)
    z = h
    for _ in range(K):
        part = prop(u, sidx, didx)
        z, u = pl.pallas_call(
            _combine_body,
            out_shape=[jax.ShapeDtypeStruct((n_pad, nhid), f32),
                       jax.ShapeDtypeStruct((n_pad, nhid), f32)])(
            part, u, h, dinv)

    v = pl.pallas_call(
        _att_body,
        out_shape=jax.ShapeDtypeStruct((n_pad, nclass), f32))(
        z, ltp, g, attW1, ab1r, attw2, Wg, dinv)

    partf = _make_prop(n_pad, bpt, nclass)(v, sidx, didx)
    out = pl.pallas_call(
        _final_body,
        out_shape=jax.ShapeDtypeStruct((n_pad, nclass), f32))(
        partf, v, dinv, bgr)
    return out[:n]


# speedup vs baseline: 24.8690x; 2.4541x over previous
import functools

import jax
import jax.numpy as jnp
from jax import lax
from jax.experimental import pallas as pl
from jax.experimental.pallas import tpu as pltpu
from jax.experimental.pallas import tpu_sc as plsc

ALPHA = 0.1
K = 10
NC, NS = 2, 16
NW = NC * NS
EB = 128
DEPTH = 4


def _mesh():
    return plsc.VectorSubcoreMesh(
        core_axis_name="c", subcore_axis_name="s",
        num_cores=NC, num_subcores=NS)


_SC_PARAMS = pltpu.CompilerParams(needs_layout_passes=False,
                                  use_tc_tiling_on_sc=False)


def _edge_pass(u_sh, acc_sh, sid, sidx_h, didx_h, sidx_v, didx_v,
               rows, semg, sems, bps):
    hb = bps // 2

    def eb(i, c):
        b0 = i * DEPTH
        for j in range(DEPTH):
            b = b0 + j
            pltpu.make_async_copy(
                u_sh.at[sidx_v.at[b]], rows[j], semg[j]).wait()
            pltpu.async_copy(
                rows[j], acc_sh.at[didx_v.at[b]], sems[j], add=True)
        for j in range(DEPTH):
            b = b0 + j
            pltpu.make_async_copy(
                rows[j], acc_sh.at[didx_v.at[b]], sems[j]).wait()
            nb = b0 + DEPTH + j

            @pl.when(nb < hb)
            def _():
                pltpu.async_copy(u_sh.at[sidx_v.at[nb]], rows[j], semg[j])
        return c

    for ph in range(2):
        pltpu.sync_copy(sidx_h.at[sid, pl.ds(ph * hb, hb)], sidx_v)
        pltpu.sync_copy(didx_h.at[sid, pl.ds(ph * hb, hb)], didx_v)
        for j in range(DEPTH):
            pltpu.async_copy(u_sh.at[sidx_v.at[j]], rows[j], semg[j])
        lax.fori_loop(0, hb // DEPTH, eb, 0)


@functools.lru_cache(maxsize=None)
def _make_sweep(n_pad, bps, hw, n_iter):
    rps = n_pad // NS
    zr = rps // 4

    def body(u0c_h, hc_h, dinv_h, sidx_h, didx_h, zc_h,
             sidx_v, didx_v, h_ch, dinv_ch, acc_b, zb, acc_sh, u_sh, *bufs):
        rows = bufs[:DEPTH]
        semg = bufs[DEPTH:2 * DEPTH]
        sems = bufs[2 * DEPTH:3 * DEPTH]
        cid = lax.axis_index("c")
        sid = lax.axis_index("s")
        r0 = sid * rps
        chunk = pl.ds(r0, rps)
        pltpu.sync_copy(hc_h.at[cid, chunk], h_ch)
        pltpu.sync_copy(dinv_h.at[chunk], dinv_ch)
        pltpu.sync_copy(u0c_h.at[cid, chunk], u_sh.at[chunk])
        zero = jnp.zeros((16,), jnp.float32)

        def zzb(i, c):
            for g in range(hw // 16):
                zb[i, pl.ds(g * 16, 16)] = zero
            return c
        lax.fori_loop(0, zr, zzb, 0)
        for q in range(4):
            pltpu.sync_copy(zb, acc_sh.at[pl.ds(r0 + q * zr, zr)])
        plsc.subcore_barrier()

        def it_body(it, c):
            _edge_pass(u_sh, acc_sh, sid, sidx_h, didx_h, sidx_v, didx_v,
                       rows, semg, sems, bps)
            plsc.subcore_barrier()
            pltpu.sync_copy(acc_sh.at[chunk], acc_b)
            for q in range(4):
                pltpu.sync_copy(zb, acc_sh.at[pl.ds(r0 + q * zr, zr)])

            def p1(r16, cc):
                dv16 = dinv_ch[pl.ds(r16 * 16, 16)]
                for k in range(16):
                    r = r16 * 16 + k
                    dva = (1.0 - ALPHA) * dv16[k]
                    for g in range(hw // 16):
                        sl = pl.ds(g * 16, 16)
                        acc_b[r, sl] = dva * acc_b[r, sl] \
                            + ALPHA * h_ch[r, sl]
                return cc
            lax.fori_loop(0, rps // 16, p1, 0)

            @pl.when(it == n_iter - 1)
            def _():
                pltpu.sync_copy(acc_b, zc_h.at[cid, chunk])

            def p2(r16, cc):
                dv16 = dinv_ch[pl.ds(r16 * 16, 16)]
                for k in range(16):
                    r = r16 * 16 + k
                    dv = dv16[k]
                    for g in range(hw // 16):
                        sl = pl.ds(g * 16, 16)
                        acc_b[r, sl] = dv * acc_b[r, sl]
                return cc
            lax.fori_loop(0, rps // 16, p2, 0)
            pltpu.sync_copy(acc_b, u_sh.at[chunk])
            plsc.subcore_barrier()
            return c
        lax.fori_loop(0, n_iter, it_body, 0)

    return pl.kernel(
        body,
        out_type=jax.ShapeDtypeStruct((NC, n_pad, hw), jnp.float32),
        mesh=_mesh(),
        compiler_params=_SC_PARAMS,
        scratch_types=[
            pltpu.VMEM((bps // 2, EB), jnp.int32),
            pltpu.VMEM((bps // 2, EB), jnp.int32),
            pltpu.VMEM((rps, hw), jnp.float32),
            pltpu.VMEM((rps,), jnp.float32),
            pltpu.VMEM((rps, hw), jnp.float32),
            pltpu.VMEM((zr, hw), jnp.float32),
            pltpu.VMEM_SHARED((n_pad, hw), jnp.float32),
            pltpu.VMEM_SHARED((n_pad, hw), jnp.float32),
        ] + [pltpu.VMEM((EB, hw), jnp.float32)] * DEPTH
          + [pltpu.SemaphoreType.DMA] * (2 * DEPTH),
    )


@functools.lru_cache(maxsize=None)
def _make_fprop(n_pad, bps, hw):
    rps = n_pad // NS
    zr = rps // 4

    def body(vc_h, sidx_h, didx_h, out_h,
             sidx_v, didx_v, zb, acc_sh, u_sh, *bufs):
        rows = bufs[:DEPTH]
        semg = bufs[DEPTH:2 * DEPTH]
        sems = bufs[2 * DEPTH:3 * DEPTH]
        cid = lax.axis_index("c")
        sid = lax.axis_index("s")
        r0 = sid * rps
        chunk = pl.ds(r0, rps)
        pltpu.sync_copy(vc_h.at[cid, chunk], u_sh.at[chunk])
        zero = jnp.zeros((16,), jnp.float32)

        def zzb(i, c):
            for g in range(hw // 16):
                zb[i, pl.ds(g * 16, 16)] = zero
            return c
        lax.fori_loop(0, zr, zzb, 0)
        for q in range(4):
            pltpu.sync_copy(zb, acc_sh.at[pl.ds(r0 + q * zr, zr)])
        plsc.subcore_barrier()
        _edge_pass(u_sh, acc_sh, sid, sidx_h, didx_h, sidx_v, didx_v,
                   rows, semg, sems, bps)
        plsc.subcore_barrier()
        pltpu.sync_copy(acc_sh.at[chunk], out_h.at[cid, chunk])

    return pl.kernel(
        body,
        out_type=jax.ShapeDtypeStruct((NC, n_pad, hw), jnp.float32),
        mesh=_mesh(),
        compiler_params=_SC_PARAMS,
        scratch_types=[
            pltpu.VMEM((bps // 2, EB), jnp.int32),
            pltpu.VMEM((bps // 2, EB), jnp.int32),
            pltpu.VMEM((zr, hw), jnp.float32),
            pltpu.VMEM_SHARED((n_pad, hw), jnp.float32),
            pltpu.VMEM_SHARED((n_pad, hw), jnp.float32),
        ] + [pltpu.VMEM((EB, hw), jnp.float32)] * DEPTH
          + [pltpu.SemaphoreType.DMA] * (2 * DEPTH),
    )


@functools.lru_cache(maxsize=None)
def _make_deg(n_pad, bps):
    hbps = bps // 2

    def body(didx_hbm, degp_hbm, didx_v, deg_v):
        cid = lax.axis_index("c")
        sid = lax.axis_index("s")
        wid = sid * NC + cid
        pltpu.sync_copy(didx_hbm.at[sid, pl.ds(cid * hbps, hbps)], didx_v)
        zero = jnp.zeros((16,), jnp.float32)

        def zb(i, c):
            deg_v[pl.ds(i * 16, 16)] = zero
            return c
        lax.fori_loop(0, n_pad // 16, zb, 0)
        ones = jnp.ones((16,), jnp.float32)

        def eb(b, c):
            for g in range(EB // 16):
                idx = didx_v[b, pl.ds(g * 16, 16)]
                plsc.addupdate_scatter(deg_v, [idx], ones)
            return c
        lax.fori_loop(0, hbps, eb, 0)
        pltpu.sync_copy(deg_v, degp_hbm.at[wid])

    return pl.kernel(
        body,
        out_type=jax.ShapeDtypeStruct((NW, n_pad), jnp.float32),
        mesh=_mesh(),
        compiler_params=_SC_PARAMS,
        scratch_types=[
            pltpu.VMEM((hbps, EB), jnp.int32),
            pltpu.VMEM((n_pad,), jnp.float32),
        ],
    )


def _mm1_body(x_ref, w_ref, b_ref, hc_ref):
    h = jnp.maximum(
        jnp.dot(x_ref[...], w_ref[...], preferred_element_type=jnp.float32)
        + b_ref[...], 0.0)
    half = h.shape[1] // 2
    hc_ref[0, :, :] = h[:, :half]
    hc_ref[1, :, :] = h[:, half:]


def _dinv_body(degt_ref, hc_ref, dinv_ref, u0c_ref, *, n_real):
    deg = jnp.sum(degt_ref[...], axis=1, keepdims=True)
    dinv = lax.rsqrt(jnp.maximum(deg, 1.0))
    row = lax.broadcasted_iota(jnp.int32, dinv.shape, 0)
    dinv = jnp.where(row < n_real, dinv, 0.0)
    dinv_ref[...] = dinv
    u0c_ref[...] = dinv[None, :, :] * hc_ref[...]


def _att_body(zc_ref, lt_ref, g_ref, aw1_ref, ab1_ref, aw2_ref, wg_ref,
              dinv_ref, vc_ref):
    z = jnp.concatenate([zc_ref[0, :, :], zc_ref[1, :, :]], axis=1)
    lt = lt_ref[...]
    aw1 = aw1_ref[...]
    ab1 = ab1_ref[...]
    aw2 = aw2_ref[...]
    wz = jnp.dot(jnp.tanh(
        jnp.dot(z, aw1, preferred_element_type=jnp.float32) + ab1),
        aw2, preferred_element_type=jnp.float32)
    wl = jnp.dot(jnp.tanh(
        jnp.dot(lt, aw1, preferred_element_type=jnp.float32) + ab1),
        aw2, preferred_element_type=jnp.float32)
    m = jnp.maximum(wz, wl)
    ez = jnp.exp(wz - m)
    el = jnp.exp(wl - m)
    emb2 = (ez * z + el * lt) / (ez + el)
    z2 = emb2 * g_ref[...]
    v = dinv_ref[...] * jnp.dot(
        z2, wg_ref[...], preferred_element_type=jnp.float32)
    half = v.shape[1] // 2
    vc_ref[0, :, :] = v[:, :half]
    vc_ref[1, :, :] = v[:, half:]


def _final_body(accf_ref, dinv_ref, bg_ref, out_ref):
    acc = jnp.concatenate([accf_ref[0, :, :], accf_ref[1, :, :]], axis=1)
    o = dinv_ref[...] * acc + bg_ref[...]
    m = jnp.max(o, axis=1, keepdims=True)
    s = jnp.sum(jnp.exp(o - m), axis=1, keepdims=True)
    out_ref[...] = (o - m) - jnp.log(s)


def kernel(x, edge_index, local_topo, global_topo, W1, b1,
           attW1, attb1, attw2, Wg, bg):
    n, nfeat = x.shape
    e = edge_index.shape[1]
    nhid = W1.shape[1]
    nclass = Wg.shape[1]
    n_pad = -(-(n + 1) // (NS * 16)) * NS * 16
    e_full = e + n
    bps = -(-e_full // (NS * EB))
    bps = -(-bps // (2 * DEPTH)) * 2 * DEPTH
    e_pad = NS * EB * bps

    loop = jnp.arange(n, dtype=jnp.int32)
    fill = jnp.full((e_pad - e_full,), n, jnp.int32)
    sidx = jnp.concatenate([edge_index[0], loop, fill]).reshape(NS, bps, EB)
    didx = jnp.concatenate([edge_index[1], loop, fill]).reshape(NS, bps, EB)

    pad_n = n_pad - n
    xp = jnp.pad(x, ((0, pad_n), (0, 0)))
    ltp = jnp.pad(local_topo, ((0, pad_n), (0, 0)))
    b1r = b1.reshape(1, nhid)
    ab1r = attb1.reshape(1, -1)
    bgr = bg.reshape(1, nclass)
    g = global_topo.reshape(1, nhid)
    hw_s = nhid // NC
    hw_f = nclass // NC

    f32 = jnp.float32
    hc = pl.pallas_call(
        _mm1_body,
        out_shape=jax.ShapeDtypeStruct((NC, n_pad, hw_s), f32))(xp, W1, b1r)

    degp = _make_deg(n_pad, bps)(didx)
    dinv, u0c = pl.pallas_call(
        functools.partial(_dinv_body, n_real=n),
        out_shape=[jax.ShapeDtypeStruct((n_pad, 1), f32),
                   jax.ShapeDtypeStruct((NC, n_pad, hw_s), f32)])(degp.T, hc)
    dinv_flat = dinv[:, 0]

    zc = _make_sweep(n_pad, bps, hw_s, K)(u0c, hc, dinv_flat, sidx, didx)

    vc = pl.pallas_call(
        _att_body,
        out_shape=jax.ShapeDtypeStruct((NC, n_pad, hw_f), f32))(
        zc, ltp, g, attW1, ab1r, attw2, Wg, dinv)

    accf = _make_fprop(n_pad, bps, hw_f)(vc, sidx, didx)
    out = pl.pallas_call(
        _final_body,
        out_shape=jax.ShapeDtypeStruct((n_pad, nclass), f32))(
        accf, dinv, bgr)
    return out[:n]


# speedup vs baseline: 25.6825x; 1.0327x over previous
import functools

import jax
import jax.numpy as jnp
from jax import lax
from jax.experimental import pallas as pl
from jax.experimental.pallas import tpu as pltpu
from jax.experimental.pallas import tpu_sc as plsc

ALPHA = 0.1
K = 10
NC, NS = 2, 16
NW = NC * NS
EB = 128
DEPTH = 4


def _mesh():
    return plsc.VectorSubcoreMesh(
        core_axis_name="c", subcore_axis_name="s",
        num_cores=NC, num_subcores=NS)


_SC_PARAMS = pltpu.CompilerParams(needs_layout_passes=False,
                                  use_tc_tiling_on_sc=False)


def _edge_pass(u_sh, acc_sh, sid, sidx_h, didx_h, sidx_v, didx_v,
               rows, semg, sems, bps):
    hb = bps // 2

    def eb(i, c):
        b0 = i * DEPTH
        for j in range(DEPTH):
            b = b0 + j
            pltpu.make_async_copy(
                u_sh.at[sidx_v.at[b]], rows[j], semg[j]).wait()
            pltpu.async_copy(
                rows[j], acc_sh.at[didx_v.at[b]], sems[j], add=True)
        for j in range(DEPTH):
            b = b0 + j
            pltpu.make_async_copy(
                rows[j], acc_sh.at[didx_v.at[b]], sems[j]).wait()
            nb = b0 + DEPTH + j

            @pl.when(nb < hb)
            def _():
                pltpu.async_copy(u_sh.at[sidx_v.at[nb]], rows[j], semg[j])
        return c

    for ph in range(2):
        pltpu.sync_copy(sidx_h.at[sid, pl.ds(ph * hb, hb)], sidx_v)
        pltpu.sync_copy(didx_h.at[sid, pl.ds(ph * hb, hb)], didx_v)
        for j in range(DEPTH):
            pltpu.async_copy(u_sh.at[sidx_v.at[j]], rows[j], semg[j])
        lax.fori_loop(0, hb // DEPTH, eb, 0)


def _scale_rows(dst, src, mult_ch, rps, hw):
    def p(r16, cc):
        m16 = mult_ch[pl.ds(r16 * 16, 16)]
        for k in range(16):
            r = r16 * 16 + k
            m = m16[k]
            for g in range(hw // 16):
                sl = pl.ds(g * 16, 16)
                dst[r, sl] = m * src[r, sl]
        return cc
    lax.fori_loop(0, rps // 16, p, 0)


@functools.lru_cache(maxsize=None)
def _make_sweep(n_pad, bps, hw, n_iter):
    rps = n_pad // NS

    def body(hc_h, dinv_h, sidx_h, didx_h, zc_h,
             sidx_v, didx_v, acc_b, s_ch, dinv_ch, dva_ch, dvq_ch,
             acc_sh, u_sh, *bufs):
        rows = bufs[:DEPTH]
        semg = bufs[DEPTH:2 * DEPTH]
        sems = bufs[2 * DEPTH:3 * DEPTH]
        cid = lax.axis_index("c")
        sid = lax.axis_index("s")
        r0 = sid * rps
        chunk = pl.ds(r0, rps)
        pltpu.sync_copy(hc_h.at[cid, chunk], s_ch)
        pltpu.sync_copy(dinv_h.at[chunk], dinv_ch)

        def prep(r16, cc):
            sl0 = pl.ds(r16 * 16, 16)
            dv16 = dinv_ch[sl0]
            dva16 = (1.0 - ALPHA) * dv16
            dva_ch[sl0] = dva16
            dvq_ch[sl0] = dv16 * dva16
            ss16 = jnp.where(dv16 > 0.0,
                             (ALPHA / (1.0 - ALPHA)) / dv16, 0.0)
            for k in range(16):
                r = r16 * 16 + k
                dv = dv16[k]
                ss = ss16[k]
                for g in range(hw // 16):
                    sl = pl.ds(g * 16, 16)
                    h = s_ch[r, sl]
                    acc_b[r, sl] = dv * h
                    s_ch[r, sl] = ss * h
            return cc
        lax.fori_loop(0, rps // 16, prep, 0)
        pltpu.sync_copy(acc_b, u_sh.at[chunk])
        pltpu.sync_copy(s_ch, acc_sh.at[chunk])
        plsc.subcore_barrier()

        def it_body(it, c):
            _edge_pass(u_sh, acc_sh, sid, sidx_h, didx_h, sidx_v, didx_v,
                       rows, semg, sems, bps)
            plsc.subcore_barrier()
            pltpu.sync_copy(acc_sh.at[chunk], acc_b)
            pltpu.sync_copy(s_ch, acc_sh.at[chunk])

            @pl.when(it < n_iter - 1)
            def _():
                _scale_rows(acc_b, acc_b, dvq_ch, rps, hw)
                pltpu.sync_copy(acc_b, u_sh.at[chunk])

            @pl.when(it == n_iter - 1)
            def _():
                _scale_rows(acc_b, acc_b, dva_ch, rps, hw)
                pltpu.sync_copy(acc_b, zc_h.at[cid, chunk])
            plsc.subcore_barrier()
            return c
        lax.fori_loop(0, n_iter, it_body, 0)

    return pl.kernel(
        body,
        out_type=jax.ShapeDtypeStruct((NC, n_pad, hw), jnp.float32),
        mesh=_mesh(),
        compiler_params=_SC_PARAMS,
        scratch_types=[
            pltpu.VMEM((bps // 2, EB), jnp.int32),
            pltpu.VMEM((bps // 2, EB), jnp.int32),
            pltpu.VMEM((rps, hw), jnp.float32),
            pltpu.VMEM((rps, hw), jnp.float32),
            pltpu.VMEM((rps,), jnp.float32),
            pltpu.VMEM((rps,), jnp.float32),
            pltpu.VMEM((rps,), jnp.float32),
            pltpu.VMEM_SHARED((n_pad, hw), jnp.float32),
            pltpu.VMEM_SHARED((n_pad, hw), jnp.float32),
        ] + [pltpu.VMEM((EB, hw), jnp.float32)] * DEPTH
          + [pltpu.SemaphoreType.DMA] * (2 * DEPTH),
    )


@functools.lru_cache(maxsize=None)
def _make_fprop(n_pad, bps, hw):
    rps = n_pad // NS
    zr = rps // 4

    def body(yc_h, dinv_h, sidx_h, didx_h, out_h,
             sidx_v, didx_v, buf, dinv_ch, zb, acc_sh, u_sh, *bufs):
        rows = bufs[:DEPTH]
        semg = bufs[DEPTH:2 * DEPTH]
        sems = bufs[2 * DEPTH:3 * DEPTH]
        cid = lax.axis_index("c")
        sid = lax.axis_index("s")
        r0 = sid * rps
        chunk = pl.ds(r0, rps)
        pltpu.sync_copy(yc_h.at[cid, chunk], buf)
        pltpu.sync_copy(dinv_h.at[chunk], dinv_ch)
        _scale_rows(buf, buf, dinv_ch, rps, hw)
        pltpu.sync_copy(buf, u_sh.at[chunk])
        zero = jnp.zeros((16,), jnp.float32)

        def zzb(i, c):
            for g in range(hw // 16):
                zb[i, pl.ds(g * 16, 16)] = zero
            return c
        lax.fori_loop(0, zr, zzb, 0)
        for q in range(4):
            pltpu.sync_copy(zb, acc_sh.at[pl.ds(r0 + q * zr, zr)])
        plsc.subcore_barrier()
        _edge_pass(u_sh, acc_sh, sid, sidx_h, didx_h, sidx_v, didx_v,
                   rows, semg, sems, bps)
        plsc.subcore_barrier()
        pltpu.sync_copy(acc_sh.at[chunk], buf)
        _scale_rows(buf, buf, dinv_ch, rps, hw)
        pltpu.sync_copy(buf, out_h.at[cid, chunk])

    return pl.kernel(
        body,
        out_type=jax.ShapeDtypeStruct((NC, n_pad, hw), jnp.float32),
        mesh=_mesh(),
        compiler_params=_SC_PARAMS,
        scratch_types=[
            pltpu.VMEM((bps // 2, EB), jnp.int32),
            pltpu.VMEM((bps // 2, EB), jnp.int32),
            pltpu.VMEM((rps, hw), jnp.float32),
            pltpu.VMEM((rps,), jnp.float32),
            pltpu.VMEM((zr, hw), jnp.float32),
            pltpu.VMEM_SHARED((n_pad, hw), jnp.float32),
            pltpu.VMEM_SHARED((n_pad, hw), jnp.float32),
        ] + [pltpu.VMEM((EB, hw), jnp.float32)] * DEPTH
          + [pltpu.SemaphoreType.DMA] * (2 * DEPTH),
    )


@functools.lru_cache(maxsize=None)
def _make_deg(n_pad, bps):
    hbps = bps // 2

    def body(didx_hbm, degp_hbm, didx_v, deg_v):
        cid = lax.axis_index("c")
        sid = lax.axis_index("s")
        wid = sid * NC + cid
        pltpu.sync_copy(didx_hbm.at[sid, pl.ds(cid * hbps, hbps)], didx_v)
        zero = jnp.zeros((16,), jnp.float32)

        def zb(i, c):
            deg_v[pl.ds(i * 16, 16)] = zero
            return c
        lax.fori_loop(0, n_pad // 16, zb, 0)
        ones = jnp.ones((16,), jnp.float32)

        def eb(b, c):
            for g in range(EB // 16):
                idx = didx_v[b, pl.ds(g * 16, 16)]
                plsc.addupdate_scatter(deg_v, [idx], ones)
            return c
        lax.fori_loop(0, hbps, eb, 0)
        pltpu.sync_copy(deg_v, degp_hbm.at[wid])

    return pl.kernel(
        body,
        out_type=jax.ShapeDtypeStruct((NW, n_pad), jnp.float32),
        mesh=_mesh(),
        compiler_params=_SC_PARAMS,
        scratch_types=[
            pltpu.VMEM((hbps, EB), jnp.int32),
            pltpu.VMEM((n_pad,), jnp.float32),
        ],
    )


def _mm1_body(x_ref, w_ref, b_ref, hc_ref):
    h = jnp.maximum(
        jnp.dot(x_ref[...], w_ref[...], preferred_element_type=jnp.float32)
        + b_ref[...], 0.0)
    half = h.shape[1] // 2
    hc_ref[0, :, :] = h[:, :half]
    hc_ref[1, :, :] = h[:, half:]


def _dinv_body(degp_ref, dinv_ref, *, n_real):
    deg = jnp.sum(degp_ref[...], axis=0, keepdims=True)
    dinv = lax.rsqrt(jnp.maximum(deg, 1.0))
    col = lax.broadcasted_iota(jnp.int32, dinv.shape, 1)
    dinv_ref[...] = jnp.where(col < n_real, dinv, 0.0)


def _att_body(zc_ref, lt_ref, g_ref, aw1_ref, ab1_ref, aw2_ref, wg_ref,
              yc_ref):
    z = jnp.concatenate([zc_ref[0, :, :], zc_ref[1, :, :]], axis=1)
    lt = lt_ref[...]
    aw1 = aw1_ref[...]
    ab1 = ab1_ref[...]
    aw2 = aw2_ref[...]
    wz = jnp.dot(jnp.tanh(
        jnp.dot(z, aw1, preferred_element_type=jnp.float32) + ab1),
        aw2, preferred_element_type=jnp.float32)
    wl = jnp.dot(jnp.tanh(
        jnp.dot(lt, aw1, preferred_element_type=jnp.float32) + ab1),
        aw2, preferred_element_type=jnp.float32)
    m = jnp.maximum(wz, wl)
    ez = jnp.exp(wz - m)
    el = jnp.exp(wl - m)
    emb2 = (ez * z + el * lt) / (ez + el)
    z2 = emb2 * g_ref[...]
    y = jnp.dot(z2, wg_ref[...], preferred_element_type=jnp.float32)
    half = y.shape[1] // 2
    yc_ref[0, :, :] = y[:, :half]
    yc_ref[1, :, :] = y[:, half:]


def _final_body(accf_ref, bg_ref, out_ref):
    acc = jnp.concatenate([accf_ref[0, :, :], accf_ref[1, :, :]], axis=1)
    o = acc + bg_ref[...]
    m = jnp.max(o, axis=1, keepdims=True)
    s = jnp.sum(jnp.exp(o - m), axis=1, keepdims=True)
    out_ref[...] = (o - m) - jnp.log(s)


def kernel(x, edge_index, local_topo, global_topo, W1, b1,
           attW1, attb1, attw2, Wg, bg):
    n, nfeat = x.shape
    e = edge_index.shape[1]
    nhid = W1.shape[1]
    nclass = Wg.shape[1]
    n_pad = -(-(n + 1) // (NS * 16)) * NS * 16
    e_full = e + n
    bps = -(-e_full // (NS * EB))
    bps = -(-bps // (2 * DEPTH)) * 2 * DEPTH
    e_pad = NS * EB * bps

    loop = jnp.arange(n, dtype=jnp.int32)
    fill = jnp.full((e_pad - e_full,), n, jnp.int32)
    sidx = jnp.concatenate([edge_index[0], loop, fill]).reshape(NS, bps, EB)
    didx = jnp.concatenate([edge_index[1], loop, fill]).reshape(NS, bps, EB)

    pad_n = n_pad - n
    xp = jnp.pad(x, ((0, pad_n), (0, 0)))
    ltp = jnp.pad(local_topo, ((0, pad_n), (0, 0)))
    b1r = b1.reshape(1, nhid)
    ab1r = attb1.reshape(1, -1)
    bgr = bg.reshape(1, nclass)
    g = global_topo.reshape(1, nhid)
    hw_s = nhid // NC
    hw_f = nclass // NC

    f32 = jnp.float32
    hc = pl.pallas_call(
        _mm1_body,
        out_shape=jax.ShapeDtypeStruct((NC, n_pad, hw_s), f32))(xp, W1, b1r)

    degp = _make_deg(n_pad, bps)(didx)
    dinv = pl.pallas_call(
        functools.partial(_dinv_body, n_real=n),
        out_shape=jax.ShapeDtypeStruct((1, n_pad), f32))(degp)
    dinv_flat = dinv.reshape(n_pad)

    zc = _make_sweep(n_pad, bps, hw_s, K)(hc, dinv_flat, sidx, didx)

    yc = pl.pallas_call(
        _att_body,
        out_shape=jax.ShapeDtypeStruct((NC, n_pad, hw_f), f32))(
        zc, ltp, g, attW1, ab1r, attw2, Wg)

    accf = _make_fprop(n_pad, bps, hw_f)(yc, dinv_flat, sidx, didx)
    out = pl.pallas_call(
        _final_body,
        out_shape=jax.ShapeDtypeStruct((n_pad, nclass), f32))(accf, bgr)
    return out[:n]


# speedup vs baseline: 25.7774x; 1.0037x over previous
import functools

import jax
import jax.numpy as jnp
from jax import lax
from jax.experimental import pallas as pl
from jax.experimental.pallas import tpu as pltpu
from jax.experimental.pallas import tpu_sc as plsc

ALPHA = 0.1
K = 10
NC, NS = 2, 16
NW = NC * NS
EB = 128
DEPTH = 4


def _mesh():
    return plsc.VectorSubcoreMesh(
        core_axis_name="c", subcore_axis_name="s",
        num_cores=NC, num_subcores=NS)


_SC_PARAMS = pltpu.CompilerParams(needs_layout_passes=False,
                                  use_tc_tiling_on_sc=False)


def _edge_pass(u_sh, acc_sh, sidx_v, didx_v, rows, semg, sems, bps):
    for j in range(DEPTH):
        pltpu.async_copy(u_sh.at[sidx_v.at[j]], rows[j], semg[j])

    def eb(i, c):
        b0 = i * DEPTH
        for j in range(DEPTH):
            b = b0 + j
            pltpu.make_async_copy(
                u_sh.at[sidx_v.at[b]], rows[j], semg[j]).wait()
            pltpu.async_copy(
                rows[j], acc_sh.at[didx_v.at[b]], sems[j], add=True)
        for j in range(DEPTH):
            b = b0 + j
            pltpu.make_async_copy(
                rows[j], acc_sh.at[didx_v.at[b]], sems[j]).wait()
            nb = b0 + DEPTH + j

            @pl.when(nb < bps)
            def _():
                pltpu.async_copy(u_sh.at[sidx_v.at[nb]], rows[j], semg[j])
        return c
    lax.fori_loop(0, bps // DEPTH, eb, 0)


def _scale_rows(dst, src, mult_ch, rps, hw):
    def p(r16, cc):
        m16 = mult_ch[pl.ds(r16 * 16, 16)]
        for k in range(16):
            r = r16 * 16 + k
            m = m16[k]
            for g in range(hw // 16):
                sl = pl.ds(g * 16, 16)
                dst[r, sl] = m * src[r, sl]
        return cc
    lax.fori_loop(0, rps // 16, p, 0)


@functools.lru_cache(maxsize=None)
def _make_sweep(n_pad, bps, hw, n_iter):
    rps = n_pad // NS

    def body(u0c_h, seedc_h, dinv_h, sidx_h, didx_h, zc_h,
             sidx_v, didx_v, acc_b, dinv_ch, dva_ch, dvq_ch,
             acc_sh, u_sh, *bufs):
        rows = bufs[:DEPTH]
        semg = bufs[DEPTH:2 * DEPTH]
        sems = bufs[2 * DEPTH:3 * DEPTH]
        cid = lax.axis_index("c")
        sid = lax.axis_index("s")
        r0 = sid * rps
        chunk = pl.ds(r0, rps)
        pltpu.sync_copy(sidx_h.at[sid], sidx_v)
        pltpu.sync_copy(didx_h.at[sid], didx_v)
        pltpu.sync_copy(dinv_h.at[chunk], dinv_ch)
        pltpu.sync_copy(u0c_h.at[cid, chunk], u_sh.at[chunk])
        pltpu.sync_copy(seedc_h.at[cid, chunk], acc_sh.at[chunk])

        def prep(r16, cc):
            sl0 = pl.ds(r16 * 16, 16)
            dv16 = dinv_ch[sl0]
            dva16 = (1.0 - ALPHA) * dv16
            dva_ch[sl0] = dva16
            dvq_ch[sl0] = dv16 * dva16
            return cc
        lax.fori_loop(0, rps // 16, prep, 0)
        plsc.subcore_barrier()

        def it_body(it, c):
            _edge_pass(u_sh, acc_sh, sidx_v, didx_v,
                       rows, semg, sems, bps)
            plsc.subcore_barrier()
            pltpu.sync_copy(acc_sh.at[chunk], acc_b)
            pltpu.sync_copy(seedc_h.at[cid, chunk], acc_sh.at[chunk])

            @pl.when(it < n_iter - 1)
            def _():
                _scale_rows(acc_b, acc_b, dvq_ch, rps, hw)
                pltpu.sync_copy(acc_b, u_sh.at[chunk])

            @pl.when(it == n_iter - 1)
            def _():
                _scale_rows(acc_b, acc_b, dva_ch, rps, hw)
                pltpu.sync_copy(acc_b, zc_h.at[cid, chunk])
            plsc.subcore_barrier()
            return c
        lax.fori_loop(0, n_iter, it_body, 0)

    return pl.kernel(
        body,
        out_type=jax.ShapeDtypeStruct((NC, n_pad, hw), jnp.float32),
        mesh=_mesh(),
        compiler_params=_SC_PARAMS,
        scratch_types=[
            pltpu.VMEM((bps, EB), jnp.int32),
            pltpu.VMEM((bps, EB), jnp.int32),
            pltpu.VMEM((rps, hw), jnp.float32),
            pltpu.VMEM((rps,), jnp.float32),
            pltpu.VMEM((rps,), jnp.float32),
            pltpu.VMEM((rps,), jnp.float32),
            pltpu.VMEM_SHARED((n_pad, hw), jnp.float32),
            pltpu.VMEM_SHARED((n_pad, hw), jnp.float32),
        ] + [pltpu.VMEM((EB, hw), jnp.float32)] * DEPTH
          + [pltpu.SemaphoreType.DMA] * (2 * DEPTH),
    )


@functools.lru_cache(maxsize=None)
def _make_fprop(n_pad, bps, hw):
    rps = n_pad // NS
    zr = rps // 4

    def body(yc_h, dinv_h, sidx_h, didx_h, out_h,
             sidx_v, didx_v, buf, dinv_ch, zb, acc_sh, u_sh, *bufs):
        rows = bufs[:DEPTH]
        semg = bufs[DEPTH:2 * DEPTH]
        sems = bufs[2 * DEPTH:3 * DEPTH]
        cid = lax.axis_index("c")
        sid = lax.axis_index("s")
        r0 = sid * rps
        chunk = pl.ds(r0, rps)
        pltpu.sync_copy(sidx_h.at[sid], sidx_v)
        pltpu.sync_copy(didx_h.at[sid], didx_v)
        pltpu.sync_copy(yc_h.at[cid, chunk], buf)
        pltpu.sync_copy(dinv_h.at[chunk], dinv_ch)
        _scale_rows(buf, buf, dinv_ch, rps, hw)
        pltpu.sync_copy(buf, u_sh.at[chunk])
        zero = jnp.zeros((16,), jnp.float32)

        def zzb(i, c):
            for g in range(hw // 16):
                zb[i, pl.ds(g * 16, 16)] = zero
            return c
        lax.fori_loop(0, zr, zzb, 0)
        for q in range(4):
            pltpu.sync_copy(zb, acc_sh.at[pl.ds(r0 + q * zr, zr)])
        plsc.subcore_barrier()
        _edge_pass(u_sh, acc_sh, sidx_v, didx_v, rows, semg, sems, bps)
        plsc.subcore_barrier()
        pltpu.sync_copy(acc_sh.at[chunk], buf)
        _scale_rows(buf, buf, dinv_ch, rps, hw)
        pltpu.sync_copy(buf, out_h.at[cid, chunk])

    return pl.kernel(
        body,
        out_type=jax.ShapeDtypeStruct((NC, n_pad, hw), jnp.float32),
        mesh=_mesh(),
        compiler_params=_SC_PARAMS,
        scratch_types=[
            pltpu.VMEM((bps, EB), jnp.int32),
            pltpu.VMEM((bps, EB), jnp.int32),
            pltpu.VMEM((rps, hw), jnp.float32),
            pltpu.VMEM((rps,), jnp.float32),
            pltpu.VMEM((zr, hw), jnp.float32),
            pltpu.VMEM_SHARED((n_pad, hw), jnp.float32),
            pltpu.VMEM_SHARED((n_pad, hw), jnp.float32),
        ] + [pltpu.VMEM((EB, hw), jnp.float32)] * DEPTH
          + [pltpu.SemaphoreType.DMA] * (2 * DEPTH),
    )


@functools.lru_cache(maxsize=None)
def _make_deg(n_pad, bps):
    hbps = bps // 2

    def body(didx_hbm, degp_hbm, didx_v, deg_v):
        cid = lax.axis_index("c")
        sid = lax.axis_index("s")
        wid = sid * NC + cid
        pltpu.sync_copy(didx_hbm.at[sid, pl.ds(cid * hbps, hbps)], didx_v)
        zero = jnp.zeros((16,), jnp.float32)

        def zb(i, c):
            deg_v[pl.ds(i * 16, 16)] = zero
            return c
        lax.fori_loop(0, n_pad // 16, zb, 0)
        ones = jnp.ones((16,), jnp.float32)

        def eb(b, c):
            for g in range(EB // 16):
                idx = didx_v[b, pl.ds(g * 16, 16)]
                plsc.addupdate_scatter(deg_v, [idx], ones)
            return c
        lax.fori_loop(0, hbps, eb, 0)
        pltpu.sync_copy(deg_v, degp_hbm.at[wid])

    return pl.kernel(
        body,
        out_type=jax.ShapeDtypeStruct((NW, n_pad), jnp.float32),
        mesh=_mesh(),
        compiler_params=_SC_PARAMS,
        scratch_types=[
            pltpu.VMEM((hbps, EB), jnp.int32),
            pltpu.VMEM((n_pad,), jnp.float32),
        ],
    )


def _mm1_body(x_ref, w_ref, b_ref, hc_ref):
    h = jnp.maximum(
        jnp.dot(x_ref[...], w_ref[...], preferred_element_type=jnp.float32)
        + b_ref[...], 0.0)
    half = h.shape[1] // 2
    hc_ref[0, :, :] = h[:, :half]
    hc_ref[1, :, :] = h[:, half:]


def _dinv_body(degt_ref, hc_ref, dinv_ref, u0c_ref, seedc_ref, *, n_real):
    deg = jnp.sum(degt_ref[...], axis=1, keepdims=True)
    dinv = lax.rsqrt(jnp.maximum(deg, 1.0))
    row = lax.broadcasted_iota(jnp.int32, dinv.shape, 0)
    dinv = jnp.where(row < n_real, dinv, 0.0)
    dinv_ref[...] = dinv
    hc = hc_ref[...]
    d3 = dinv[None, :, :]
    u0c_ref[...] = d3 * hc
    seedc_ref[...] = jnp.where(
        d3 > 0.0, (ALPHA / (1.0 - ALPHA)) * hc / jnp.maximum(d3, 1e-30),
        0.0)


def _att_body(zc_ref, lt_ref, g_ref, aw1_ref, ab1_ref, aw2_ref, wg_ref,
              yc_ref):
    z = jnp.concatenate([zc_ref[0, :, :], zc_ref[1, :, :]], axis=1)
    lt = lt_ref[...]
    aw1 = aw1_ref[...]
    ab1 = ab1_ref[...]
    aw2 = aw2_ref[...]
    wz = jnp.dot(jnp.tanh(
        jnp.dot(z, aw1, preferred_element_type=jnp.float32) + ab1),
        aw2, preferred_element_type=jnp.float32)
    wl = jnp.dot(jnp.tanh(
        jnp.dot(lt, aw1, preferred_element_type=jnp.float32) + ab1),
        aw2, preferred_element_type=jnp.float32)
    m = jnp.maximum(wz, wl)
    ez = jnp.exp(wz - m)
    el = jnp.exp(wl - m)
    emb2 = (ez * z + el * lt) / (ez + el)
    z2 = emb2 * g_ref[...]
    y = jnp.dot(z2, wg_ref[...], preferred_element_type=jnp.float32)
    half = y.shape[1] // 2
    yc_ref[0, :, :] = y[:, :half]
    yc_ref[1, :, :] = y[:, half:]


def _final_body(accf_ref, bg_ref, out_ref):
    acc = jnp.concatenate([accf_ref[0, :, :], accf_ref[1, :, :]], axis=1)
    o = acc + bg_ref[...]
    m = jnp.max(o, axis=1, keepdims=True)
    s = jnp.sum(jnp.exp(o - m), axis=1, keepdims=True)
    out_ref[...] = (o - m) - jnp.log(s)


def kernel(x, edge_index, local_topo, global_topo, W1, b1,
           attW1, attb1, attw2, Wg, bg):
    n, nfeat = x.shape
    e = edge_index.shape[1]
    nhid = W1.shape[1]
    nclass = Wg.shape[1]
    n_pad = -(-(n + 1) // (NS * 16)) * NS * 16
    e_full = e + n
    bps = -(-e_full // (NS * EB))
    bps = -(-bps // (2 * DEPTH)) * 2 * DEPTH
    e_pad = NS * EB * bps

    loop = jnp.arange(n, dtype=jnp.int32)
    fill = jnp.full((e_pad - e_full,), n, jnp.int32)
    sidx = jnp.concatenate([edge_index[0], loop, fill]).reshape(NS, bps, EB)
    didx = jnp.concatenate([edge_index[1], loop, fill]).reshape(NS, bps, EB)

    pad_n = n_pad - n
    xp = jnp.pad(x, ((0, pad_n), (0, 0)))
    ltp = jnp.pad(local_topo, ((0, pad_n), (0, 0)))
    b1r = b1.reshape(1, nhid)
    ab1r = attb1.reshape(1, -1)
    bgr = bg.reshape(1, nclass)
    g = global_topo.reshape(1, nhid)
    hw_s = nhid // NC
    hw_f = nclass // NC

    f32 = jnp.float32
    hc = pl.pallas_call(
        _mm1_body,
        out_shape=jax.ShapeDtypeStruct((NC, n_pad, hw_s), f32))(xp, W1, b1r)

    degp = _make_deg(n_pad, bps)(didx)
    dinv, u0c, seedc = pl.pallas_call(
        functools.partial(_dinv_body, n_real=n),
        out_shape=[jax.ShapeDtypeStruct((n_pad, 1), f32),
                   jax.ShapeDtypeStruct((NC, n_pad, hw_s), f32),
                   jax.ShapeDtypeStruct((NC, n_pad, hw_s), f32)])(
        degp.T, hc)
    dinv_flat = dinv[:, 0]

    zc = _make_sweep(n_pad, bps, hw_s, K)(u0c, seedc, dinv_flat, sidx, didx)

    yc = pl.pallas_call(
        _att_body,
        out_shape=jax.ShapeDtypeStruct((NC, n_pad, hw_f), f32))(
        zc, ltp, g, attW1, ab1r, attw2, Wg)

    accf = _make_fprop(n_pad, bps, hw_f)(yc, dinv_flat, sidx, didx)
    out = pl.pallas_call(
        _final_body,
        out_shape=jax.ShapeDtypeStruct((n_pad, nclass), f32))(accf, bgr)
    return out[:n]
